# Initial kernel scaffold; baseline (speedup 1.0000x reference)
import jax
import jax.numpy as jnp
from jax.experimental import pallas as pl


def kernel(x, mask_idxs, pos_positions, pos_changes):
    raise NotImplementedError("write your pallas kernel here")


# speedup vs baseline: 1.2103x; 1.2103x over previous
import jax
import jax.numpy as jnp
from jax import lax
from jax.experimental import pallas as pl
from jax.experimental.pallas import tpu as pltpu

_B, _S, _D, _P = 4, 4096, 2048, 16
_BS = 512


def _patch_copy_body(mask_ref, pos_ref, chg_ref, x_ref, o_ref):
    b = pl.program_id(0)
    sblk = pl.program_id(1)
    m = mask_ref[b]
    xb = x_ref[0]
    s_iota = lax.broadcasted_iota(jnp.int32, (_BS, 1), 0) + sblk * _BS
    rowsel = (s_iota == m).astype(jnp.float32)
    row = jnp.sum(xb * rowsel, axis=0, keepdims=True)
    d_iota = lax.broadcasted_iota(jnp.int32, (1, _D), 1)
    delta_row = jnp.zeros((1, _D), jnp.float32)
    for p in range(_P):
        pos_p = pos_ref[b, p]
        onehot = d_iota == pos_p
        val_p = jnp.sum(jnp.where(onehot, row, 0.0))
        delta_row = delta_row + jnp.where(
            onehot, chg_ref[b, p] * jnp.sign(val_p), 0.0
        )
    o_ref[0] = xb + rowsel * delta_row


def kernel(x, mask_idxs, pos_positions, pos_changes):
    grid = (_B, _S // _BS)
    return pl.pallas_call(
        _patch_copy_body,
        grid=grid,
        in_specs=[
            pl.BlockSpec(memory_space=pltpu.SMEM),
            pl.BlockSpec(memory_space=pltpu.SMEM),
            pl.BlockSpec(memory_space=pltpu.SMEM),
            pl.BlockSpec((1, _BS, _D), lambda b, s: (b, s, 0)),
        ],
        out_specs=pl.BlockSpec((1, _BS, _D), lambda b, s: (b, s, 0)),
        out_shape=jax.ShapeDtypeStruct((_B, _S, _D), jnp.float32),
        compiler_params=pltpu.CompilerParams(
            dimension_semantics=("parallel", "parallel"),
        ),
    )(mask_idxs, pos_positions, pos_changes, x)


# speedup vs baseline: 1.3132x; 1.0851x over previous
import jax
import jax.numpy as jnp
from jax import lax
from jax.experimental import pallas as pl
from jax.experimental.pallas import tpu as pltpu

_B, _S, _D, _P = 4, 4096, 2048, 16
_BS = 1024


def _patch_copy_body(mask_ref, pos_ref, chg_ref, x_ref, o_ref):
    b = pl.program_id(0)
    sblk = pl.program_id(1)
    m = mask_ref[b]
    o_ref[0] = x_ref[0]

    @pl.when(m // _BS == sblk)
    def _patch():
        r = m - sblk * _BS
        row = x_ref[0, pl.ds(r, 1), :]
        d_iota = lax.broadcasted_iota(jnp.int32, (1, _D), 1)
        delta_row = jnp.zeros((1, _D), jnp.float32)
        for p in range(_P):
            pos_p = pos_ref[b, p]
            onehot = d_iota == pos_p
            val_p = jnp.sum(jnp.where(onehot, row, 0.0))
            delta_row = delta_row + jnp.where(
                onehot, chg_ref[b, p] * jnp.sign(val_p), 0.0
            )
        o_ref[0, pl.ds(r, 1), :] = row + delta_row


def kernel(x, mask_idxs, pos_positions, pos_changes):
    grid = (_B, _S // _BS)
    return pl.pallas_call(
        _patch_copy_body,
        grid=grid,
        in_specs=[
            pl.BlockSpec(memory_space=pltpu.SMEM),
            pl.BlockSpec(memory_space=pltpu.SMEM),
            pl.BlockSpec(memory_space=pltpu.SMEM),
            pl.BlockSpec((1, _BS, _D), lambda b, s: (b, s, 0)),
        ],
        out_specs=pl.BlockSpec((1, _BS, _D), lambda b, s: (b, s, 0)),
        out_shape=jax.ShapeDtypeStruct((_B, _S, _D), jnp.float32),
        compiler_params=pltpu.CompilerParams(
            dimension_semantics=("parallel", "parallel"),
        ),
    )(mask_idxs, pos_positions, pos_changes, x)
